# Initial kernel scaffold; baseline (speedup 1.0000x reference)
import jax
import jax.numpy as jnp
from jax.experimental import pallas as pl


def kernel(x, groups, W_pred, b_pred):
    raise NotImplementedError("write your pallas kernel here")


# speedup vs baseline: 10.2948x; 10.2948x over previous
import dataclasses

import jax
import jax.numpy as jnp
from jax import lax
from jax.experimental import pallas as pl
from jax.experimental.pallas import tpu as pltpu
from jax.experimental.pallas import tpu_sc as plsc

B, C, H, W = 16, 3, 224, 224
G = 256
MC = 8
P = H * W
HALF = P // 2
CHUNK = 3136
NLANE = 16
NC, NS = 2, 16

_MESH = plsc.VectorSubcoreMesh(core_axis_name="core", subcore_axis_name="subcore")

_CP = pltpu.CompilerParams()
if "needs_layout_passes" in pltpu.CompilerParams.__dataclass_fields__:
    _CP = dataclasses.replace(_CP, needs_layout_passes=False)


def _worker_id():
    return lax.axis_index("core") * NS + lax.axis_index("subcore")


def _sigmoid(v):
    return 1.0 / (1.0 + jnp.exp(-v))


def _k1_body(x_hbm, g_hbm, w_hbm, pp_hbm, part_hbm,
             x0v, x1v, x2v, gv, ppv, sums, counts, wv, sem):
    wid = _worker_id()
    b = wid // 2
    base = (wid % 2) * HALF

    pltpu.sync_copy(w_hbm, wv)
    w0 = wv[pl.ds(0, NLANE)]
    w1 = wv[pl.ds(NLANE, NLANE)]
    w2 = wv[pl.ds(2 * NLANE, NLANE)]
    bias = wv[pl.ds(3 * NLANE, NLANE)]

    zero = jnp.zeros((NLANE,), jnp.float32)
    ones = jnp.full((NLANE,), 1.0, jnp.float32)

    @pl.loop(0, G, step=NLANE)
    def _(g):
        sums[pl.ds(g, NLANE)] = zero
        counts[pl.ds(g, NLANE)] = zero

    @pl.loop(0, HALF, step=CHUNK)
    def _(off):
        start = b * P + base + off
        xstart = b * C * P + base + off
        pltpu.async_copy(x_hbm.at[pl.ds(xstart, CHUNK)], x0v, sem).wait()
        pltpu.async_copy(x_hbm.at[pl.ds(xstart + P, CHUNK)], x1v, sem).wait()
        pltpu.async_copy(x_hbm.at[pl.ds(xstart + 2 * P, CHUNK)], x2v, sem).wait()
        pltpu.async_copy(g_hbm.at[pl.ds(start, CHUNK)], gv, sem).wait()

        @pl.loop(0, CHUNK, step=NLANE)
        def _(i):
            sl = pl.ds(i, NLANE)
            e = x0v[sl] * w0 + x1v[sl] * w1 + x2v[sl] * w2 + bias
            ppv[sl] = _sigmoid(e)
            g = gv[sl]
            plsc.addupdate_scatter(sums, [g], e)
            plsc.addupdate_scatter(counts, [g], ones)

        pltpu.async_copy(ppv, pp_hbm.at[pl.ds(start, CHUNK)], sem).wait()

    pltpu.async_copy(sums, part_hbm.at[pl.ds(wid * 2 * G, G)], sem).wait()
    pltpu.async_copy(counts, part_hbm.at[pl.ds(wid * 2 * G + G, G)], sem).wait()


def _k2_body(g_hbm, part_hbm, l_hbm, mask_hbm, gp_hbm,
             pa, pb, lv, gpv, table, gv, selv, sem):
    wid = _worker_id()
    b = wid // 2
    half = wid % 2
    base = half * HALF

    pltpu.async_copy(part_hbm.at[pl.ds((2 * b) * 2 * G, 2 * G)], pa, sem).wait()
    pltpu.async_copy(part_hbm.at[pl.ds((2 * b + 1) * 2 * G, 2 * G)], pb, sem).wait()
    pltpu.async_copy(l_hbm.at[pl.ds(b * MC * G, MC * G)], lv, sem).wait()

    iota = lax.iota(jnp.int32, NLANE)
    iota8 = iota * MC

    @pl.loop(0, G, step=NLANE)
    def _(g):
        sl = pl.ds(g, NLANE)
        s = pa[sl] + pb[sl]
        n = pa[pl.ds(G + g, NLANE)] + pb[pl.ds(G + g, NLANE)]
        mu = s / jnp.maximum(n, 1.0)
        gpv[sl] = _sigmoid(mu)
        for m in range(MC):
            hard = jnp.where(mu + lv[pl.ds(m * G + g, NLANE)] > 0.0, 1.0, 0.0)
            plsc.store_scatter(table, [iota8 + (g * MC + m)], hard)

    @pl.when(half == 0)
    def _():
        pltpu.sync_copy(gpv, gp_hbm.at[pl.ds(b * G, G)])

    @pl.loop(0, HALF, step=CHUNK)
    def _(off):
        start = base + off
        pltpu.async_copy(g_hbm.at[pl.ds(b * P + start, CHUNK)], gv, sem).wait()

        @pl.loop(0, CHUNK, step=NLANE)
        def _(i):
            g8 = gv[pl.ds(i, NLANE)] * MC
            st = iota8 + i * MC
            for m in range(MC):
                vals = plsc.load_gather(table, [g8 + m])
                plsc.store_scatter(selv, [st + m], vals)

        mbase = (b * C * P + start) * MC
        c0 = pltpu.async_copy(selv, mask_hbm.at[pl.ds(mbase, CHUNK * MC)], sem)
        c1 = pltpu.async_copy(selv, mask_hbm.at[pl.ds(mbase + P * MC, CHUNK * MC)], sem)
        c2 = pltpu.async_copy(selv, mask_hbm.at[pl.ds(mbase + 2 * P * MC, CHUNK * MC)], sem)
        c0.wait()
        c1.wait()
        c2.wait()


def kernel(x, groups, W_pred, b_pred):
    x1d = x.reshape(B * C * P)
    g1d = groups.reshape(B * P)

    wvec = jnp.concatenate([W_pred[0], b_pred[0:1]])
    wflat = jnp.broadcast_to(wvec[:, None], (4, NLANE)).reshape(4 * NLANE)

    u = jax.random.uniform(jax.random.key(42), (B, G, MC),
                           minval=1e-6, maxval=1.0 - 1e-6)
    lnoise = jnp.log(u) - jnp.log1p(-u)
    lflat = lnoise.transpose(0, 2, 1).reshape(B * MC * G)

    k1 = pl.kernel(
        _k1_body,
        out_type=[
            jax.ShapeDtypeStruct((B * P,), jnp.float32),
            jax.ShapeDtypeStruct((NC * NS * 2 * G,), jnp.float32),
        ],
        mesh=_MESH,
        compiler_params=_CP,
        scratch_types=[
            pltpu.VMEM((CHUNK,), jnp.float32),
            pltpu.VMEM((CHUNK,), jnp.float32),
            pltpu.VMEM((CHUNK,), jnp.float32),
            pltpu.VMEM((CHUNK,), jnp.int32),
            pltpu.VMEM((CHUNK,), jnp.float32),
            pltpu.VMEM((G,), jnp.float32),
            pltpu.VMEM((G,), jnp.float32),
            pltpu.VMEM((4 * NLANE,), jnp.float32),
            pltpu.SemaphoreType.DMA,
        ],
    )
    pp, partials = k1(x1d, g1d, wflat)

    k2 = pl.kernel(
        _k2_body,
        out_type=[
            jax.ShapeDtypeStruct((B * C * P * MC,), jnp.float32),
            jax.ShapeDtypeStruct((B * G,), jnp.float32),
        ],
        mesh=_MESH,
        compiler_params=_CP,
        scratch_types=[
            pltpu.VMEM((2 * G,), jnp.float32),
            pltpu.VMEM((2 * G,), jnp.float32),
            pltpu.VMEM((MC * G,), jnp.float32),
            pltpu.VMEM((G,), jnp.float32),
            pltpu.VMEM((G * MC,), jnp.float32),
            pltpu.VMEM((CHUNK,), jnp.int32),
            pltpu.VMEM((CHUNK * MC,), jnp.float32),
            pltpu.SemaphoreType.DMA,
        ],
    )
    maskflat, group_probs = k2(g1d, partials, lflat)

    mask = maskflat.reshape(B, C, H, W, MC)
    pixel_probs = pp.reshape(B, H, W)
    return (mask, group_probs.reshape(B, G), pixel_probs)


# speedup vs baseline: 63.7985x; 6.1972x over previous
import dataclasses

import jax
import jax.numpy as jnp
from jax import lax
from jax.experimental import pallas as pl
from jax.experimental.pallas import tpu as pltpu
from jax.experimental.pallas import tpu_sc as plsc

B, C, H, W = 16, 3, 224, 224
G = 256
MC = 8
P = H * W
HROWS = H // 2
HB = 16
NLANE = 16
NC, NS = 2, 16

_MESH = plsc.VectorSubcoreMesh(core_axis_name="core", subcore_axis_name="subcore")

_CP = pltpu.CompilerParams(use_tc_tiling_on_sc=True)
if "needs_layout_passes" in pltpu.CompilerParams.__dataclass_fields__:
    _CP = dataclasses.replace(_CP, needs_layout_passes=False)


def _worker_id():
    return lax.axis_index("core") * NS + lax.axis_index("subcore")


def _sigmoid(v):
    return 1.0 / (1.0 + jnp.exp(-v))


def _k1_body(x_hbm, g_hbm, w_hbm, pp_hbm, part_hbm,
             x0v, x1v, x2v, gv, ppv, sums, counts, wv, sem):
    wid = _worker_id()
    b = wid // 2
    hbase = (wid % 2) * HROWS

    pltpu.sync_copy(w_hbm, wv)
    w0 = wv[pl.ds(0, NLANE)]
    w1 = wv[pl.ds(NLANE, NLANE)]
    w2 = wv[pl.ds(2 * NLANE, NLANE)]
    bias = wv[pl.ds(3 * NLANE, NLANE)]

    zero = jnp.zeros((NLANE,), jnp.float32)
    ones = jnp.full((NLANE,), 1.0, jnp.float32)

    @pl.loop(0, G, step=NLANE)
    def _(g):
        sums[pl.ds(g, NLANE)] = zero
        counts[pl.ds(g, NLANE)] = zero

    @pl.loop(0, HROWS, step=HB)
    def _(hb):
        h0 = hbase + hb
        pltpu.async_copy(x_hbm.at[b, 0, pl.ds(h0, HB)], x0v, sem).wait()
        pltpu.async_copy(x_hbm.at[b, 1, pl.ds(h0, HB)], x1v, sem).wait()
        pltpu.async_copy(x_hbm.at[b, 2, pl.ds(h0, HB)], x2v, sem).wait()
        pltpu.async_copy(g_hbm.at[b, pl.ds(h0, HB)], gv, sem).wait()

        @pl.loop(0, HB)
        def _(r):
            @pl.loop(0, W, step=NLANE)
            def _(w):
                sl = (r, pl.ds(w, NLANE))
                e = x0v[sl] * w0 + x1v[sl] * w1 + x2v[sl] * w2 + bias
                ppv[sl] = _sigmoid(e)
                g = gv[sl]
                plsc.addupdate_scatter(sums, [g], e)
                plsc.addupdate_scatter(counts, [g], ones)

        pltpu.async_copy(ppv, pp_hbm.at[b, pl.ds(h0, HB)], sem).wait()

    pltpu.async_copy(sums, part_hbm.at[pl.ds(wid * 2 * G, G)], sem).wait()
    pltpu.async_copy(counts, part_hbm.at[pl.ds(wid * 2 * G + G, G)], sem).wait()


def _k2_body(g_hbm, part_hbm, l_hbm, mask_hbm, gp_hbm,
             pa, pb, lv, gpv, table, gv, selv, sem):
    wid = _worker_id()
    b = wid // 2
    half = wid % 2
    hbase = half * HROWS

    pltpu.async_copy(part_hbm.at[pl.ds((2 * b) * 2 * G, 2 * G)], pa, sem).wait()
    pltpu.async_copy(part_hbm.at[pl.ds((2 * b + 1) * 2 * G, 2 * G)], pb, sem).wait()
    pltpu.async_copy(l_hbm.at[pl.ds(b * MC * G, MC * G)], lv, sem).wait()

    iota = lax.iota(jnp.int32, NLANE)
    iota8 = iota * MC

    @pl.loop(0, G, step=NLANE)
    def _(g):
        sl = pl.ds(g, NLANE)
        s = pa[sl] + pb[sl]
        n = pa[pl.ds(G + g, NLANE)] + pb[pl.ds(G + g, NLANE)]
        mu = s / jnp.maximum(n, 1.0)
        gpv[sl] = _sigmoid(mu)
        for m in range(MC):
            hard = jnp.where(mu + lv[pl.ds(m * G + g, NLANE)] > 0.0, 1.0, 0.0)
            plsc.store_scatter(table, [iota8 + (g * MC + m)], hard)

    @pl.when(half == 0)
    def _():
        pltpu.sync_copy(gpv, gp_hbm.at[pl.ds(b * G, G)])

    @pl.loop(0, HROWS, step=HB)
    def _(hb):
        h0 = hbase + hb
        pltpu.async_copy(g_hbm.at[b, pl.ds(h0, HB)], gv, sem).wait()

        @pl.loop(0, HB)
        def _(r):
            @pl.loop(0, W, step=NLANE)
            def _(w):
                g8 = gv[r, pl.ds(w, NLANE)] * MC
                for m in range(MC):
                    selv[r, m, pl.ds(w, NLANE)] = plsc.load_gather(
                        table, [g8 + m])

        c0 = pltpu.async_copy(selv, mask_hbm.at[b, 0, pl.ds(h0, HB)], sem)
        c1 = pltpu.async_copy(selv, mask_hbm.at[b, 1, pl.ds(h0, HB)], sem)
        c2 = pltpu.async_copy(selv, mask_hbm.at[b, 2, pl.ds(h0, HB)], sem)
        c0.wait()
        c1.wait()
        c2.wait()


def kernel(x, groups, W_pred, b_pred):
    wvec = jnp.concatenate([W_pred[0], b_pred[0:1]])
    wflat = jnp.broadcast_to(wvec[:, None], (4, NLANE)).reshape(4 * NLANE)

    u = jax.random.uniform(jax.random.key(42), (B, G, MC),
                           minval=1e-6, maxval=1.0 - 1e-6)
    lnoise = jnp.log(u) - jnp.log1p(-u)
    lflat = lnoise.transpose(0, 2, 1).reshape(B * MC * G)

    k1 = pl.kernel(
        _k1_body,
        out_type=[
            jax.ShapeDtypeStruct((B, H, W), jnp.float32),
            jax.ShapeDtypeStruct((NC * NS * 2 * G,), jnp.float32),
        ],
        mesh=_MESH,
        compiler_params=_CP,
        scratch_types=[
            pltpu.VMEM((HB, W), jnp.float32),
            pltpu.VMEM((HB, W), jnp.float32),
            pltpu.VMEM((HB, W), jnp.float32),
            pltpu.VMEM((HB, W), jnp.int32),
            pltpu.VMEM((HB, W), jnp.float32),
            pltpu.VMEM((G,), jnp.float32),
            pltpu.VMEM((G,), jnp.float32),
            pltpu.VMEM((4 * NLANE,), jnp.float32),
            pltpu.SemaphoreType.DMA,
        ],
    )
    pp, partials = k1(x, groups, wflat)

    k2 = pl.kernel(
        _k2_body,
        out_type=[
            jax.ShapeDtypeStruct((B, C, H, MC, W), jnp.float32),
            jax.ShapeDtypeStruct((B * G,), jnp.float32),
        ],
        mesh=_MESH,
        compiler_params=_CP,
        scratch_types=[
            pltpu.VMEM((2 * G,), jnp.float32),
            pltpu.VMEM((2 * G,), jnp.float32),
            pltpu.VMEM((MC * G,), jnp.float32),
            pltpu.VMEM((G,), jnp.float32),
            pltpu.VMEM((G * MC,), jnp.float32),
            pltpu.VMEM((HB, W), jnp.int32),
            pltpu.VMEM((HB, MC, W), jnp.float32),
            pltpu.SemaphoreType.DMA,
        ],
    )
    maskT, group_probs = k2(groups, partials, lflat)

    mask = maskT.transpose(0, 1, 2, 4, 3)
    return (mask, group_probs.reshape(B, G), pp)


# speedup vs baseline: 81.9191x; 1.2840x over previous
import dataclasses

import jax
import jax.numpy as jnp
from jax import lax
from jax.experimental import pallas as pl
from jax.experimental.pallas import tpu as pltpu
from jax.experimental.pallas import tpu_sc as plsc

B, C, H, W = 16, 3, 224, 224
G = 256
MC = 8
P = H * W
HROWS = H // 2
HB = 16
NLANE = 16
NC, NS = 2, 16

_MESH = plsc.VectorSubcoreMesh(core_axis_name="core", subcore_axis_name="subcore")

_CP = pltpu.CompilerParams(use_tc_tiling_on_sc=True)
if "needs_layout_passes" in pltpu.CompilerParams.__dataclass_fields__:
    _CP = dataclasses.replace(_CP, needs_layout_passes=False)


def _worker_id():
    return lax.axis_index("core") * NS + lax.axis_index("subcore")


def _sigmoid(v):
    return 1.0 / (1.0 + jnp.exp(-v))


NBANDS = HROWS // HB


def _k1_body(x_hbm, g_hbm, w_hbm, pp_hbm, part_hbm,
             xv0, xv1, gv0, gv1, pv0, pv1, sums, counts, wv,
             semx, semg, semp):
    wid = _worker_id()
    b = wid // 2
    hbase = (wid % 2) * HROWS
    xvs, gvs, pvs = (xv0, xv1), (gv0, gv1), (pv0, pv1)

    pltpu.sync_copy(w_hbm, wv)
    w0 = wv[pl.ds(0, NLANE)]
    w1 = wv[pl.ds(NLANE, NLANE)]
    w2 = wv[pl.ds(2 * NLANE, NLANE)]
    bias = wv[pl.ds(3 * NLANE, NLANE)]

    zero = jnp.zeros((NLANE,), jnp.float32)
    ones = jnp.full((NLANE,), 1.0, jnp.float32)

    @pl.loop(0, G, step=NLANE)
    def _(g):
        sums[pl.ds(g, NLANE)] = zero
        counts[pl.ds(g, NLANE)] = zero

    def fetch(k):
        h0 = hbase + k * HB
        xc = pltpu.async_copy(x_hbm.at[b, :, pl.ds(h0, HB)], xvs[k % 2], semx)
        gc = pltpu.async_copy(g_hbm.at[b, pl.ds(h0, HB)], gvs[k % 2], semg)
        return xc, gc

    pend = fetch(0)
    ppcop = [None] * NBANDS
    for k in range(NBANDS):
        xc, gc = pend
        xc.wait()
        gc.wait()
        if k + 1 < NBANDS:
            pend = fetch(k + 1)
        if k >= 2:
            ppcop[k - 2].wait()
        xv, gv, ppv = xvs[k % 2], gvs[k % 2], pvs[k % 2]

        @pl.loop(0, HB)
        def _(r):
            @pl.loop(0, W, step=NLANE)
            def _(w):
                sl = (r, pl.ds(w, NLANE))
                e = (xv[0, r, pl.ds(w, NLANE)] * w0
                     + xv[1, r, pl.ds(w, NLANE)] * w1
                     + xv[2, r, pl.ds(w, NLANE)] * w2 + bias)
                ppv[sl] = _sigmoid(e)
                g = gv[sl]
                plsc.addupdate_scatter(sums, [g], e)
                plsc.addupdate_scatter(counts, [g], ones)

        h0 = hbase + k * HB
        ppcop[k] = pltpu.async_copy(ppv, pp_hbm.at[b, pl.ds(h0, HB)], semp)

    ppcop[NBANDS - 2].wait()
    ppcop[NBANDS - 1].wait()
    pltpu.async_copy(sums, part_hbm.at[pl.ds(wid * 2 * G, G)], semx).wait()
    pltpu.async_copy(counts, part_hbm.at[pl.ds(wid * 2 * G + G, G)], semx).wait()


def _k2_body(g_hbm, part_hbm, l_hbm, mask_hbm, gp_hbm,
             pa, pb, lv, gpv, table, gv0, gv1, sv0, sv1, semg, semm):
    wid = _worker_id()
    b = wid // 2
    half = wid % 2
    hbase = half * HROWS

    ca = pltpu.async_copy(part_hbm.at[pl.ds((2 * b) * 2 * G, 2 * G)], pa, semg)
    cb = pltpu.async_copy(part_hbm.at[pl.ds((2 * b + 1) * 2 * G, 2 * G)], pb, semg)
    cl = pltpu.async_copy(l_hbm.at[pl.ds(b * MC * G, MC * G)], lv, semg)
    ca.wait()
    cb.wait()
    cl.wait()

    iota = lax.iota(jnp.int32, NLANE)
    iota8 = iota * MC

    @pl.loop(0, G, step=NLANE)
    def _(g):
        sl = pl.ds(g, NLANE)
        s = pa[sl] + pb[sl]
        n = pa[pl.ds(G + g, NLANE)] + pb[pl.ds(G + g, NLANE)]
        mu = s / jnp.maximum(n, 1.0)
        gpv[sl] = _sigmoid(mu)
        for m in range(MC):
            hard = jnp.where(mu + lv[pl.ds(m * G + g, NLANE)] > 0.0, 1.0, 0.0)
            plsc.store_scatter(table, [iota8 + (g * MC + m)], hard)

    @pl.when(half == 0)
    def _():
        pltpu.sync_copy(gpv, gp_hbm.at[pl.ds(b * G, G)])

    gvs, svs = (gv0, gv1), (sv0, sv1)

    def fetch(k):
        h0 = hbase + k * HB
        return pltpu.async_copy(g_hbm.at[b, pl.ds(h0, HB)], gvs[k % 2], semg)

    pend = fetch(0)
    mcop = [None] * NBANDS
    for k in range(NBANDS):
        pend.wait()
        if k + 1 < NBANDS:
            pend = fetch(k + 1)
        if k >= 2:
            for h in mcop[k - 2]:
                h.wait()
        gv, selv = gvs[k % 2], svs[k % 2]

        @pl.loop(0, HB)
        def _(r):
            @pl.loop(0, W, step=NLANE)
            def _(w):
                g8 = gv[r, pl.ds(w, NLANE)] * MC
                for m in range(MC):
                    selv[r, m, pl.ds(w, NLANE)] = plsc.load_gather(
                        table, [g8 + m])

        h0 = hbase + k * HB
        mcop[k] = [
            pltpu.async_copy(selv, mask_hbm.at[b, c, pl.ds(h0, HB)], semm)
            for c in range(C)
        ]

    for k in (NBANDS - 2, NBANDS - 1):
        for h in mcop[k]:
            h.wait()


def kernel(x, groups, W_pred, b_pred):
    wvec = jnp.concatenate([W_pred[0], b_pred[0:1]])
    wflat = jnp.broadcast_to(wvec[:, None], (4, NLANE)).reshape(4 * NLANE)

    u = jax.random.uniform(jax.random.key(42), (B, G, MC),
                           minval=1e-6, maxval=1.0 - 1e-6)
    lnoise = jnp.log(u) - jnp.log1p(-u)
    lflat = lnoise.transpose(0, 2, 1).reshape(B * MC * G)

    k1 = pl.kernel(
        _k1_body,
        out_type=[
            jax.ShapeDtypeStruct((B, H, W), jnp.float32),
            jax.ShapeDtypeStruct((NC * NS * 2 * G,), jnp.float32),
        ],
        mesh=_MESH,
        compiler_params=_CP,
        scratch_types=[
            pltpu.VMEM((C, HB, W), jnp.float32),
            pltpu.VMEM((C, HB, W), jnp.float32),
            pltpu.VMEM((HB, W), jnp.int32),
            pltpu.VMEM((HB, W), jnp.int32),
            pltpu.VMEM((HB, W), jnp.float32),
            pltpu.VMEM((HB, W), jnp.float32),
            pltpu.VMEM((G,), jnp.float32),
            pltpu.VMEM((G,), jnp.float32),
            pltpu.VMEM((4 * NLANE,), jnp.float32),
            pltpu.SemaphoreType.DMA,
            pltpu.SemaphoreType.DMA,
            pltpu.SemaphoreType.DMA,
        ],
    )
    pp, partials = k1(x, groups, wflat)

    k2 = pl.kernel(
        _k2_body,
        out_type=[
            jax.ShapeDtypeStruct((B, C, H, MC, W), jnp.float32),
            jax.ShapeDtypeStruct((B * G,), jnp.float32),
        ],
        mesh=_MESH,
        compiler_params=_CP,
        scratch_types=[
            pltpu.VMEM((2 * G,), jnp.float32),
            pltpu.VMEM((2 * G,), jnp.float32),
            pltpu.VMEM((MC * G,), jnp.float32),
            pltpu.VMEM((G,), jnp.float32),
            pltpu.VMEM((G * MC,), jnp.float32),
            pltpu.VMEM((HB, W), jnp.int32),
            pltpu.VMEM((HB, W), jnp.int32),
            pltpu.VMEM((HB, MC, W), jnp.float32),
            pltpu.VMEM((HB, MC, W), jnp.float32),
            pltpu.SemaphoreType.DMA,
            pltpu.SemaphoreType.DMA,
        ],
    )
    maskT, group_probs = k2(groups, partials, lflat)

    mask = maskT.transpose(0, 1, 2, 4, 3)
    return (mask, group_probs.reshape(B, G), pp)
